# baseline (speedup 1.0000x reference)
import jax
import jax.numpy as jnp
from jax.experimental import pallas as pl
from jax.experimental.pallas import tpu as pltpu
from jax.experimental.pallas import tpu_sc as plsc


def _sc_gather(table, idx_flat):
    n = idx_flat.shape[0]
    emb = table.shape[1]
    window = 128
    idx2 = idx_flat.reshape(1, n)
    mesh = plsc.VectorSubcoreMesh(core_axis_name="c", subcore_axis_name="s")

    @pl.kernel(
        out_type=jax.ShapeDtypeStruct((n, emb), table.dtype),
        mesh=mesh,
    )
    def gather_kernel(tbl_hbm, i_hbm, o_hbm):
        def body(i_vmem, o_vmem):
            pltpu.sync_copy(tbl_hbm.at[i_vmem.at[0]], o_vmem)

        pltpu.emit_pipeline(
            body,
            grid=(n // window,),
            in_specs=[pl.BlockSpec((1, window), index_map=lambda i: (0, i))],
            out_specs=[pl.BlockSpec((window, emb), index_map=lambda i: (i, 0))],
            core_axis_name=("c", "s"),
            dimension_semantics=(pltpu.PARALLEL,),
        )(i_hbm, o_hbm)

    return gather_kernel(table, idx2)


def _fc1(flat, W1, b1):
    b, d = flat.shape
    h = W1.shape[1]
    bm = 1024

    def body(f_ref, w_ref, bias_ref, o_ref):
        acc = jnp.dot(f_ref[...], w_ref[...], preferred_element_type=jnp.float32)
        o_ref[...] = jnp.maximum(acc + bias_ref[...], 0.0).astype(jnp.bfloat16)

    return pl.pallas_call(
        body,
        grid=(b // bm,),
        in_specs=[
            pl.BlockSpec((bm, d), lambda i: (i, 0)),
            pl.BlockSpec((d, h), lambda i: (0, 0)),
            pl.BlockSpec((1, h), lambda i: (0, 0)),
        ],
        out_specs=pl.BlockSpec((bm, h), lambda i: (i, 0)),
        out_shape=jax.ShapeDtypeStruct((b, h), jnp.bfloat16),
    )(flat, W1, b1.reshape(1, h))


def _fc2(h, W2, b2):
    b, hid = h.shape
    v = W2.shape[1]
    bn = 512
    nj = pl.cdiv(v, bn)

    def body(h_ref, w_ref, bias_ref, o_ref):
        w = w_ref[...].astype(jnp.bfloat16)
        acc = jnp.dot(h_ref[...], w, preferred_element_type=jnp.float32)
        o_ref[...] = acc + bias_ref[...]

    return pl.pallas_call(
        body,
        grid=(nj,),
        in_specs=[
            pl.BlockSpec((b, hid), lambda j: (0, 0)),
            pl.BlockSpec((hid, bn), lambda j: (0, j)),
            pl.BlockSpec((1, bn), lambda j: (0, j)),
        ],
        out_specs=pl.BlockSpec((b, bn), lambda j: (0, j)),
        out_shape=jax.ShapeDtypeStruct((b, v), jnp.float32),
    )(h, W2, b2.reshape(1, v))


def kernel(x, emb_table, W1, b1, W2, b2):
    batch, ctx = x.shape
    emb = emb_table.shape[1]
    flat_rows = _sc_gather(emb_table, x.reshape(batch * ctx))
    flat = flat_rows.reshape(batch, ctx * emb)
    h = _fc1(flat, W1, b1)
    return _fc2(h, W2, b2)


# speedup vs baseline: 2.1287x; 2.1287x over previous
import jax
import jax.numpy as jnp
from jax.experimental import pallas as pl
from jax.experimental.pallas import tpu as pltpu
from jax.experimental.pallas import tpu_sc as plsc


def _sc_gather(table, idx_flat):
    n = idx_flat.shape[0]
    emb = table.shape[1]
    window = 128
    idx2 = idx_flat.reshape(1, n)
    mesh = plsc.VectorSubcoreMesh(core_axis_name="c", subcore_axis_name="s")

    @pl.kernel(
        out_type=jax.ShapeDtypeStruct((n, emb), table.dtype),
        mesh=mesh,
    )
    def gather_kernel(tbl_hbm, i_hbm, o_hbm):
        def body(i_vmem, o_vmem):
            pltpu.sync_copy(tbl_hbm.at[i_vmem.at[0]], o_vmem)

        pltpu.emit_pipeline(
            body,
            grid=(n // window,),
            in_specs=[pl.BlockSpec((1, window), index_map=lambda i: (0, i))],
            out_specs=[pl.BlockSpec((window, emb), index_map=lambda i: (i, 0))],
            core_axis_name=("c", "s"),
            dimension_semantics=(pltpu.PARALLEL,),
        )(i_hbm, o_hbm)

    return gather_kernel(table, idx2)


def _fc1_t(flat, W1, b1):
    b, d = flat.shape
    h = W1.shape[1]
    bm = 1024

    def body(w_ref, f_ref, bias_ref, o_ref):
        acc = jax.lax.dot_general(
            w_ref[...], f_ref[...],
            dimension_numbers=(((0,), (1,)), ((), ())),
            preferred_element_type=jnp.float32,
        )
        o_ref[...] = jnp.maximum(acc + bias_ref[...], 0.0).astype(jnp.bfloat16)

    return pl.pallas_call(
        body,
        grid=(b // bm,),
        in_specs=[
            pl.BlockSpec((d, h), lambda i: (0, 0)),
            pl.BlockSpec((bm, d), lambda i: (i, 0)),
            pl.BlockSpec((h, 1), lambda i: (0, 0)),
        ],
        out_specs=pl.BlockSpec((h, bm), lambda i: (0, i)),
        out_shape=jax.ShapeDtypeStruct((h, b), jnp.bfloat16),
    )(W1, flat, b1.reshape(h, 1))


def _fc2_t(hT, W2t, b2):
    hid, b = hT.shape
    v = W2t.shape[0]
    bn = 512
    nj = pl.cdiv(v, bn)

    def body(w_ref, h_ref, bias_ref, o_ref):
        w = w_ref[...].astype(jnp.bfloat16)
        acc = jnp.dot(w, h_ref[...], preferred_element_type=jnp.float32)
        o_ref[...] = acc + bias_ref[...]

    return pl.pallas_call(
        body,
        grid=(nj,),
        in_specs=[
            pl.BlockSpec((bn, hid), lambda j: (j, 0)),
            pl.BlockSpec((hid, b), lambda j: (0, 0)),
            pl.BlockSpec((bn, 1), lambda j: (j, 0)),
        ],
        out_specs=pl.BlockSpec((bn, b), lambda j: (j, 0)),
        out_shape=jax.ShapeDtypeStruct((v, b), jnp.float32),
    )(W2t, hT, b2.reshape(v, 1))


def kernel(x, emb_table, W1, b1, W2, b2):
    batch, ctx = x.shape
    emb = emb_table.shape[1]
    flat_rows = _sc_gather(emb_table, x.reshape(batch * ctx))
    flat = flat_rows.reshape(batch, ctx * emb)
    hT = _fc1_t(flat, W1, b1)
    logits_t = _fc2_t(hT, W2.T, b2)
    return logits_t.T


# speedup vs baseline: 2.1468x; 1.0085x over previous
import jax
import jax.numpy as jnp
from jax.experimental import pallas as pl
from jax.experimental.pallas import tpu as pltpu
from jax.experimental.pallas import tpu_sc as plsc


def _sc_gather(table, idx_flat):
    n = idx_flat.shape[0]
    emb = table.shape[1]
    window = 128
    idx2 = idx_flat.reshape(1, n)
    mesh = plsc.VectorSubcoreMesh(core_axis_name="c", subcore_axis_name="s")

    @pl.kernel(
        out_type=jax.ShapeDtypeStruct((n, emb), table.dtype),
        mesh=mesh,
    )
    def gather_kernel(tbl_hbm, i_hbm, o_hbm):
        def body(i_vmem, o_vmem):
            pltpu.sync_copy(tbl_hbm.at[i_vmem.at[0]], o_vmem)

        pltpu.emit_pipeline(
            body,
            grid=(n // window,),
            in_specs=[pl.BlockSpec((1, window), index_map=lambda i: (0, i))],
            out_specs=[pl.BlockSpec((window, emb), index_map=lambda i: (i, 0))],
            core_axis_name=("c", "s"),
            dimension_semantics=(pltpu.PARALLEL,),
        )(i_hbm, o_hbm)

    return gather_kernel(table, idx2)


def _fc1_t(flat, W1, b1):
    b, d = flat.shape
    h = W1.shape[1]
    bm = 1024

    def body(w_ref, f_ref, bias_ref, o_ref):
        acc = jax.lax.dot_general(
            w_ref[...], f_ref[...],
            dimension_numbers=(((0,), (1,)), ((), ())),
            preferred_element_type=jnp.float32,
        )
        o_ref[...] = jnp.maximum(acc + bias_ref[...], 0.0).astype(jnp.bfloat16)

    return pl.pallas_call(
        body,
        grid=(b // bm,),
        in_specs=[
            pl.BlockSpec((d, h), lambda i: (0, 0)),
            pl.BlockSpec((bm, d), lambda i: (i, 0)),
            pl.BlockSpec((h, 1), lambda i: (0, 0)),
        ],
        out_specs=pl.BlockSpec((h, bm), lambda i: (0, i)),
        out_shape=jax.ShapeDtypeStruct((h, b), jnp.bfloat16),
    )(W1, flat, b1.reshape(h, 1))


def _fc2_t(hT, W2t, b2):
    hid, b = hT.shape
    v = W2t.shape[0]
    bn = 512
    nj = pl.cdiv(v, bn)

    def body(w_ref, h_ref, bias_ref, o_ref):
        w = w_ref[...].astype(jnp.bfloat16)
        acc = jnp.dot(w, h_ref[...], preferred_element_type=jnp.float32)
        bias_col = bias_ref[...].reshape(1, bn).T
        o_ref[...] = acc + bias_col

    return pl.pallas_call(
        body,
        grid=(nj,),
        in_specs=[
            pl.BlockSpec((bn, hid), lambda j: (j, 0)),
            pl.BlockSpec((hid, b), lambda j: (0, 0)),
            pl.BlockSpec((1, bn), lambda j: (0, j)),
        ],
        out_specs=pl.BlockSpec((bn, b), lambda j: (j, 0)),
        out_shape=jax.ShapeDtypeStruct((v, b), jnp.float32),
    )(W2t, hT, b2.reshape(1, v))


def kernel(x, emb_table, W1, b1, W2, b2):
    batch, ctx = x.shape
    emb = emb_table.shape[1]
    flat_rows = _sc_gather(emb_table, x.reshape(batch * ctx))
    flat = flat_rows.reshape(batch, ctx * emb)
    hT = _fc1_t(flat, W1, b1)
    logits_t = _fc2_t(hT, W2.T, b2)
    return logits_t.T
